# baseline (speedup 1.0000x reference)
import functools

import jax
import jax.numpy as jnp
from jax import lax
from jax.experimental import pallas as pl
from jax.experimental.pallas import tpu as pltpu
from jax.experimental.pallas import tpu_sc as plsc

_VOCAB = 1000
_C = 64
_CP = 128
_T = 8
_B = 4096

_NC = 2
_NS = 16
_NW = _NC * _NS
_ROWS = _B * _T
_RPW = _ROWS // _NW
_CHUNK = 128
_NCHUNK = _RPW // _CHUNK

_R_TC = 1024


def _sc_gather(tok_pad, idx2):
    mesh = plsc.VectorSubcoreMesh(core_axis_name="c", subcore_axis_name="s")

    @functools.partial(
        pl.kernel,
        mesh=mesh,
        out_type=jax.ShapeDtypeStruct((_NW * _NCHUNK, _CHUNK, _CP), jnp.float32),
        scratch_types=[
            pltpu.VMEM((_NCHUNK, _CHUNK), jnp.int32),
            pltpu.VMEM((2, _CHUNK, _CP), jnp.float32),
            pltpu.SemaphoreType.DMA,
            pltpu.SemaphoreType.DMA,
        ],
    )
    def k(tok_hbm, idx_hbm, out_hbm, idx_v, buf, sem0, sem1):
        wid = lax.axis_index("s") * _NC + lax.axis_index("c")
        base = wid * _NCHUNK
        pltpu.sync_copy(idx_hbm.at[pl.ds(base, _NCHUNK)], idx_v)
        sems = [sem0, sem1]
        copies = [None, None]
        copies[0] = pltpu.async_copy(
            tok_hbm.at[idx_v.at[0]], buf.at[0], sems[0])
        for j in range(_NCHUNK):
            if j + 1 < _NCHUNK:
                copies[(j + 1) % 2] = pltpu.async_copy(
                    tok_hbm.at[idx_v.at[j + 1]], buf.at[(j + 1) % 2],
                    sems[(j + 1) % 2])
            copies[j % 2].wait()
            pltpu.sync_copy(buf.at[j % 2], out_hbm.at[base + j])

    return k(tok_pad, idx2)


def _tc_body(x_ref, pos_ref, w_ref, b_ref, o_ref):
    x = x_ref[...].reshape(_R_TC // _T, _T, _CP) + pos_ref[...][None, :, :]
    y = jnp.dot(x.reshape(_R_TC, _CP), w_ref[...],
                preferred_element_type=jnp.float32)
    o_ref[...] = y + b_ref[...]


def kernel(idx, tok_table, pos_table, W, b):
    B, T = idx.shape
    tok_pad = jnp.pad(tok_table, ((0, 0), (0, _CP - _C)))
    pos_pad = jnp.pad(pos_table, ((0, 0), (0, _CP - _C)))
    W_pad = jnp.pad(W, ((0, _CP - _C), (0, 0)))
    idx2 = idx.reshape(_NW * _NCHUNK, _CHUNK)
    tok_emb = _sc_gather(tok_pad, idx2).reshape(_ROWS, _CP)

    out = pl.pallas_call(
        _tc_body,
        grid=(_ROWS // _R_TC,),
        in_specs=[
            pl.BlockSpec((_R_TC, _CP), lambda i: (i, 0)),
            pl.BlockSpec((_T, _CP), lambda i: (0, 0)),
            pl.BlockSpec((_CP, _VOCAB), lambda i: (0, 0)),
            pl.BlockSpec((1, _VOCAB), lambda i: (0, 0)),
        ],
        out_specs=pl.BlockSpec((_R_TC, _VOCAB), lambda i: (i, 0)),
        out_shape=jax.ShapeDtypeStruct((_ROWS, _VOCAB), jnp.float32),
    )(tok_emb, pos_pad, W_pad, b.reshape(1, _VOCAB))

    return out.reshape(B, T, _VOCAB)
